# Initial kernel scaffold; baseline (speedup 1.0000x reference)
import jax
import jax.numpy as jnp
from jax.experimental import pallas as pl


def kernel(x, edge_index, W1, b1, W2, b2):
    raise NotImplementedError("write your pallas kernel here")


# speedup vs baseline: 20.0174x; 20.0174x over previous
import functools

import jax
import jax.numpy as jnp
from jax import lax
from jax.experimental import pallas as pl
from jax.experimental.pallas import tpu as pltpu
from jax.experimental.pallas import tpu_sc as plsc

N = 10000
NP = 10240
D = 128
E = 320000
NC, NS = 2, 16
NW = NC * NS
EPT = E // NW
CH = 125
NCH = EPT // CH
RPT = NP // NS
ZB = 128
DEGW = 128

def _fill(ref, rows, width, value):
  vec = jnp.full((16,), value, jnp.float32)

  def body(i, _):
    for j in range(width // 16):
      ref[i, pl.ds(j * 16, 16)] = vec
    return 0

  lax.fori_loop(0, rows, body, 0)


@functools.cache
def _sc_kernels():
  mesh = plsc.VectorSubcoreMesh(
      core_axis_name="c", subcore_axis_name="s",
      num_cores=NC, num_subcores=NS)

  deg_k = functools.partial(
      pl.kernel,
      out_type=jax.ShapeDtypeStruct((NC, NP, DEGW), jnp.float32),
      mesh=mesh,
      scratch_types=[
          pltpu.VMEM((NCH, CH), jnp.int32),
          pltpu.VMEM((ZB, DEGW), jnp.float32),
          pltpu.VMEM_SHARED((NP, DEGW), jnp.float32),
      ],
  )(_deg_body)

  msg_k = functools.partial(
      pl.kernel,
      out_type=jax.ShapeDtypeStruct((NC, NP, D), jnp.float32),
      mesh=mesh,
      scratch_types=[
          pltpu.VMEM((NCH, CH), jnp.int32),
          pltpu.VMEM((NCH, CH), jnp.int32),
          pltpu.VMEM((ZB, D), jnp.float32),
          pltpu.VMEM_SHARED((NP, D), jnp.float32),
          pltpu.SemaphoreType.DMA,
      ],
  )(_msg_body)

  return deg_k, msg_k


def _deg_body(dst_hbm, out_hbm, dst_v, buf_v, acc_sh):
  cid = lax.axis_index("c")
  sid = lax.axis_index("s")
  ebase = pl.multiple_of((cid * NS + sid) * NCH, 8)
  row0 = pl.multiple_of(sid * RPT, 8)

  _fill(buf_v, ZB, DEGW, 0.0)
  for k in range(RPT // ZB):
    pltpu.sync_copy(buf_v, acc_sh.at[pl.ds(row0 + k * ZB, ZB)])
  plsc.subcore_barrier()

  pltpu.sync_copy(dst_hbm.at[pl.ds(ebase, NCH)], dst_v)
  _fill(buf_v, CH, DEGW, 1.0)

  def body(c, _):
    pltpu.sync_copy(buf_v.at[pl.ds(0, CH)], acc_sh.at[dst_v.at[c]], add=True)
    return 0

  lax.fori_loop(0, NCH, body, 0)
  plsc.subcore_barrier()

  pltpu.sync_copy(acc_sh.at[pl.ds(row0, RPT)],
                  out_hbm.at[cid, pl.ds(row0, RPT)])


def _msg_body(g_hbm, src_hbm, dst_hbm, out_hbm, src_v, dst_v, rows_v,
              acc_sh, sem):
  cid = lax.axis_index("c")
  sid = lax.axis_index("s")
  ebase = pl.multiple_of((cid * NS + sid) * NCH, 8)
  row0 = pl.multiple_of(sid * RPT, 8)

  _fill(rows_v, ZB, D, 0.0)
  for k in range(RPT // ZB):
    pltpu.sync_copy(rows_v, acc_sh.at[pl.ds(row0 + k * ZB, ZB)])
  plsc.subcore_barrier()

  pltpu.sync_copy(src_hbm.at[pl.ds(ebase, NCH)], src_v)
  pltpu.sync_copy(dst_hbm.at[pl.ds(ebase, NCH)], dst_v)

  def body(c, _):
    pltpu.async_copy(g_hbm.at[src_v.at[c]], rows_v.at[pl.ds(0, CH)],
                     sem).wait()
    pltpu.sync_copy(rows_v.at[pl.ds(0, CH)], acc_sh.at[dst_v.at[c]], add=True)
    return 0

  lax.fori_loop(0, NCH, body, 0)
  plsc.subcore_barrier()

  pltpu.sync_copy(acc_sh.at[pl.ds(row0, RPT)],
                  out_hbm.at[cid, pl.ds(row0, RPT)])


_BR = 2000
_GRID = N // _BR


def _dinv(d0, d1):
  deg = d0[:, 0:1] + d1[:, 0:1] + 1.0
  return lax.rsqrt(deg)


def _pre_body(x_ref, w_ref, d0_ref, d1_ref, o_ref):
  h = jnp.dot(x_ref[...], w_ref[...], preferred_element_type=jnp.float32)
  o_ref[...] = h * _dinv(d0_ref[...], d1_ref[...])


def _mid_body(a0_ref, a1_ref, g_ref, d0_ref, d1_ref, b_ref, w_ref, o_ref):
  dinv = _dinv(d0_ref[...], d1_ref[...])
  h = dinv * (a0_ref[...] + a1_ref[...] + g_ref[...]) + b_ref[...]
  h = jnp.maximum(h, 0.0)
  o_ref[...] = jnp.dot(h, w_ref[...],
                       preferred_element_type=jnp.float32) * dinv


def _post_body(a0_ref, a1_ref, g_ref, d0_ref, d1_ref, b_ref, o_ref):
  dinv = _dinv(d0_ref[...], d1_ref[...])
  o_ref[...] = dinv * (a0_ref[...] + a1_ref[...] + g_ref[...]) + b_ref[...]


_row_spec = pl.BlockSpec((_BR, D), lambda i: (i, 0))
_deg_spec = pl.BlockSpec((_BR, DEGW), lambda i: (i, 0))
_w_spec = pl.BlockSpec((D, D), lambda i: (0, 0))
_b_spec = pl.BlockSpec((1, D), lambda i: (0, 0))
_out_t = jax.ShapeDtypeStruct((N, D), jnp.float32)

_pre = pl.pallas_call(
    _pre_body, grid=(_GRID,),
    in_specs=[_row_spec, _w_spec, _deg_spec, _deg_spec],
    out_specs=_row_spec, out_shape=_out_t)

_mid = pl.pallas_call(
    _mid_body, grid=(_GRID,),
    in_specs=[_row_spec, _row_spec, _row_spec, _deg_spec, _deg_spec,
              _b_spec, _w_spec],
    out_specs=_row_spec, out_shape=_out_t)

_post = pl.pallas_call(
    _post_body, grid=(_GRID,),
    in_specs=[_row_spec, _row_spec, _row_spec, _deg_spec, _deg_spec, _b_spec],
    out_specs=_row_spec, out_shape=_out_t)


def kernel(x, edge_index, W1, b1, W2, b2):
  src = edge_index[0].astype(jnp.int32).reshape(NW * NCH, CH)
  dst = edge_index[1].astype(jnp.int32).reshape(NW * NCH, CH)
  b1 = b1.reshape(1, D)
  b2 = b2.reshape(1, D)

  deg_k, msg_k = _sc_kernels()
  degp = deg_k(dst)
  d0, d1 = degp[0], degp[1]
  g1 = _pre(x, W1, d0, d1)
  acc1 = msg_k(g1, src, dst)
  g2 = _mid(acc1[0], acc1[1], g1, d0, d1, b1, W2)
  acc2 = msg_k(g2, src, dst)
  return _post(acc2[0], acc2[1], g2, d0, d1, b2)


# speedup vs baseline: 27.2305x; 1.3603x over previous
import functools

import jax
import jax.numpy as jnp
from jax import lax
from jax.experimental import pallas as pl
from jax.experimental.pallas import tpu as pltpu
from jax.experimental.pallas import tpu_sc as plsc

N = 10000
NP = 10240
D = 128
E = 320000
NC, NS = 2, 16
NW = NC * NS
EPT = E // NW
CH = 125
NCH = EPT // CH
RPT = NP // NS
ZB = 128
DEGW = 128

def _fill(ref, rows, width, value):
  vec = jnp.full((16,), value, jnp.float32)

  def body(i, _):
    for j in range(width // 16):
      ref[i, pl.ds(j * 16, 16)] = vec
    return 0

  lax.fori_loop(0, rows, body, 0)


@functools.cache
def _sc_kernels():
  mesh = plsc.VectorSubcoreMesh(
      core_axis_name="c", subcore_axis_name="s",
      num_cores=NC, num_subcores=NS)

  deg_k = functools.partial(
      pl.kernel,
      out_type=jax.ShapeDtypeStruct((NC, NP, DEGW), jnp.float32),
      mesh=mesh,
      scratch_types=[
          pltpu.VMEM((NCH, CH), jnp.int32),
          pltpu.VMEM((ZB, DEGW), jnp.float32),
          pltpu.VMEM_SHARED((NP, DEGW), jnp.float32),
      ],
  )(_deg_body)

  msg_k = functools.partial(
      pl.kernel,
      out_type=jax.ShapeDtypeStruct((NC, NP, D), jnp.float32),
      mesh=mesh,
      scratch_types=[
          pltpu.VMEM((NCH // 2, CH), jnp.int32),
          pltpu.VMEM((NCH // 2, CH), jnp.int32),
          pltpu.VMEM((2 * ZB, D), jnp.float32),
          pltpu.VMEM_SHARED((NP, D), jnp.float32),
          pltpu.SemaphoreType.DMA,
          pltpu.SemaphoreType.DMA,
      ],
  )(_msg_body)

  return deg_k, msg_k


def _deg_body(dst_hbm, out_hbm, dst_v, buf_v, acc_sh):
  cid = lax.axis_index("c")
  sid = lax.axis_index("s")
  ebase = pl.multiple_of((cid * NS + sid) * NCH, 8)
  row0 = pl.multiple_of(sid * RPT, 8)

  _fill(buf_v, ZB, DEGW, 0.0)
  for k in range(RPT // ZB):
    pltpu.sync_copy(buf_v, acc_sh.at[pl.ds(row0 + k * ZB, ZB)])
  plsc.subcore_barrier()

  pltpu.sync_copy(dst_hbm.at[pl.ds(ebase, NCH)], dst_v)
  _fill(buf_v, CH, DEGW, 1.0)

  def body(c, _):
    pltpu.sync_copy(buf_v.at[pl.ds(0, CH)], acc_sh.at[dst_v.at[c]], add=True)
    return 0

  lax.fori_loop(0, NCH, body, 0)
  plsc.subcore_barrier()

  pltpu.sync_copy(acc_sh.at[pl.ds(row0, RPT)],
                  out_hbm.at[cid, pl.ds(row0, RPT)])


def _msg_body(g_hbm, src_hbm, dst_hbm, out_hbm, src_v, dst_v, rows_v,
              acc_sh, sem0, sem1):
  cid = lax.axis_index("c")
  sid = lax.axis_index("s")
  ebase = pl.multiple_of((cid * NS + sid) * NCH, 8)
  row0 = pl.multiple_of(sid * RPT, 8)

  _fill(rows_v, ZB, D, 0.0)
  for k in range(RPT // ZB):
    pltpu.sync_copy(rows_v.at[pl.ds(0, ZB)], acc_sh.at[pl.ds(row0 + k * ZB, ZB)])
  plsc.subcore_barrier()

  NH = NCH // 2
  sems = (sem0, sem1)
  bufs = (rows_v.at[pl.ds(0, CH)], rows_v.at[pl.ds(ZB, CH)])

  for half in range(2):
    pltpu.sync_copy(src_hbm.at[pl.ds(ebase + half * NH, NH)], src_v)
    pltpu.sync_copy(dst_hbm.at[pl.ds(ebase + half * NH, NH)], dst_v)
    pltpu.async_copy(g_hbm.at[src_v.at[0]], bufs[0], sems[0])
    pltpu.async_copy(g_hbm.at[src_v.at[1]], bufs[1], sems[1])

    def body(i, _):
      c = i * 2
      for b in range(2):
        pltpu.make_async_copy(g_hbm.at[src_v.at[c + b]], bufs[b],
                              sems[b]).wait()
        pltpu.sync_copy(bufs[b], acc_sh.at[dst_v.at[c + b]], add=True)
        pltpu.async_copy(g_hbm.at[src_v.at[c + b + 2]], bufs[b], sems[b])
      return 0

    lax.fori_loop(0, NH // 2 - 1, body, 0)
    for b in range(2):
      c = NH - 2 + b
      pltpu.make_async_copy(g_hbm.at[src_v.at[c]], bufs[b], sems[b]).wait()
      pltpu.sync_copy(bufs[b], acc_sh.at[dst_v.at[c]], add=True)
  plsc.subcore_barrier()

  pltpu.sync_copy(acc_sh.at[pl.ds(row0, RPT)],
                  out_hbm.at[cid, pl.ds(row0, RPT)])


_BR = 2000
_GRID = N // _BR


def _dinv(d0, d1):
  deg = d0[:, 0:1] + d1[:, 0:1] + 1.0
  return lax.rsqrt(deg)


def _pre_body(x_ref, w_ref, d0_ref, d1_ref, o_ref):
  h = jnp.dot(x_ref[...], w_ref[...], preferred_element_type=jnp.float32)
  o_ref[...] = h * _dinv(d0_ref[...], d1_ref[...])


def _mid_body(a0_ref, a1_ref, g_ref, d0_ref, d1_ref, b_ref, w_ref, o_ref):
  dinv = _dinv(d0_ref[...], d1_ref[...])
  h = dinv * (a0_ref[...] + a1_ref[...] + g_ref[...]) + b_ref[...]
  h = jnp.maximum(h, 0.0)
  o_ref[...] = jnp.dot(h, w_ref[...],
                       preferred_element_type=jnp.float32) * dinv


def _post_body(a0_ref, a1_ref, g_ref, d0_ref, d1_ref, b_ref, o_ref):
  dinv = _dinv(d0_ref[...], d1_ref[...])
  o_ref[...] = dinv * (a0_ref[...] + a1_ref[...] + g_ref[...]) + b_ref[...]


_row_spec = pl.BlockSpec((_BR, D), lambda i: (i, 0))
_deg_spec = pl.BlockSpec((_BR, DEGW), lambda i: (i, 0))
_w_spec = pl.BlockSpec((D, D), lambda i: (0, 0))
_b_spec = pl.BlockSpec((1, D), lambda i: (0, 0))
_out_t = jax.ShapeDtypeStruct((N, D), jnp.float32)

_pre = pl.pallas_call(
    _pre_body, grid=(_GRID,),
    in_specs=[_row_spec, _w_spec, _deg_spec, _deg_spec],
    out_specs=_row_spec, out_shape=_out_t)

_mid = pl.pallas_call(
    _mid_body, grid=(_GRID,),
    in_specs=[_row_spec, _row_spec, _row_spec, _deg_spec, _deg_spec,
              _b_spec, _w_spec],
    out_specs=_row_spec, out_shape=_out_t)

_post = pl.pallas_call(
    _post_body, grid=(_GRID,),
    in_specs=[_row_spec, _row_spec, _row_spec, _deg_spec, _deg_spec, _b_spec],
    out_specs=_row_spec, out_shape=_out_t)


def kernel(x, edge_index, W1, b1, W2, b2):
  src = edge_index[0].astype(jnp.int32).reshape(NW * NCH, CH)
  dst = edge_index[1].astype(jnp.int32).reshape(NW * NCH, CH)
  b1 = b1.reshape(1, D)
  b2 = b2.reshape(1, D)

  deg_k, msg_k = _sc_kernels()
  degp = deg_k(dst)
  d0, d1 = degp[0], degp[1]
  g1 = _pre(x, W1, d0, d1)
  acc1 = msg_k(g1, src, dst)
  g2 = _mid(acc1[0], acc1[1], g1, d0, d1, b1, W2)
  acc2 = msg_k(g2, src, dst)
  return _post(acc2[0], acc2[1], g2, d0, d1, b2)


# speedup vs baseline: 30.2771x; 1.1119x over previous
import functools

import jax
import jax.numpy as jnp
from jax import lax
from jax.experimental import pallas as pl
from jax.experimental.pallas import tpu as pltpu
from jax.experimental.pallas import tpu_sc as plsc

N = 10000
NP = 10240
D = 128
E = 320000
NC, NS = 2, 16
NW = NC * NS
EPT = E // NW
CH = 125
NCH = EPT // CH
RPT = NP // NS
ZB = 128
DEGW = 32

def _fill(ref, rows, width, value):
  vec = jnp.full((16,), value, jnp.float32)

  def body(i, _):
    for j in range(width // 16):
      ref[i, pl.ds(j * 16, 16)] = vec
    return 0

  lax.fori_loop(0, rows, body, 0)


@functools.cache
def _sc_kernels():
  mesh = plsc.VectorSubcoreMesh(
      core_axis_name="c", subcore_axis_name="s",
      num_cores=NC, num_subcores=NS)

  deg_k = functools.partial(
      pl.kernel,
      out_type=jax.ShapeDtypeStruct((NC, NP, DEGW), jnp.float32),
      mesh=mesh,
      scratch_types=[
          pltpu.VMEM((NCH, CH), jnp.int32),
          pltpu.VMEM((ZB, DEGW), jnp.float32),
          pltpu.VMEM_SHARED((NP, DEGW), jnp.float32),
      ],
  )(_deg_body)

  msg_k = functools.partial(
      pl.kernel,
      out_type=jax.ShapeDtypeStruct((NC, NP, D), jnp.float32),
      mesh=mesh,
      scratch_types=[
          pltpu.VMEM((NCH // 2, CH), jnp.int32),
          pltpu.VMEM((NCH // 2, CH), jnp.int32),
          pltpu.VMEM((2 * ZB, D), jnp.float32),
          pltpu.VMEM_SHARED((NP, D), jnp.float32),
          pltpu.SemaphoreType.DMA,
          pltpu.SemaphoreType.DMA,
      ],
  )(_msg_body)

  return deg_k, msg_k


def _deg_body(dst_hbm, out_hbm, dst_v, buf_v, acc_sh):
  cid = lax.axis_index("c")
  sid = lax.axis_index("s")
  ebase = pl.multiple_of((cid * NS + sid) * NCH, 8)
  row0 = pl.multiple_of(sid * RPT, 8)

  _fill(buf_v, ZB, DEGW, 0.0)
  for k in range(RPT // ZB):
    pltpu.sync_copy(buf_v, acc_sh.at[pl.ds(row0 + k * ZB, ZB)])
  plsc.subcore_barrier()

  pltpu.sync_copy(dst_hbm.at[pl.ds(ebase, NCH)], dst_v)
  _fill(buf_v, CH, DEGW, 1.0)

  def body(c, _):
    pltpu.sync_copy(buf_v.at[pl.ds(0, CH)], acc_sh.at[dst_v.at[c]], add=True)
    return 0

  lax.fori_loop(0, NCH, body, 0)
  plsc.subcore_barrier()

  pltpu.sync_copy(acc_sh.at[pl.ds(row0, RPT)],
                  out_hbm.at[cid, pl.ds(row0, RPT)])


def _msg_body(g_hbm, src_hbm, dst_hbm, out_hbm, src_v, dst_v, rows_v,
              acc_sh, sem0, sem1):
  cid = lax.axis_index("c")
  sid = lax.axis_index("s")
  ebase = pl.multiple_of((cid * NS + sid) * NCH, 8)
  row0 = pl.multiple_of(sid * RPT, 8)

  _fill(rows_v, ZB, D, 0.0)
  for k in range(RPT // ZB):
    pltpu.sync_copy(rows_v.at[pl.ds(0, ZB)], acc_sh.at[pl.ds(row0 + k * ZB, ZB)])
  plsc.subcore_barrier()

  NH = NCH // 2
  sems = (sem0, sem1)
  bufs = (rows_v.at[pl.ds(0, CH)], rows_v.at[pl.ds(ZB, CH)])

  for half in range(2):
    pltpu.sync_copy(src_hbm.at[pl.ds(ebase + half * NH, NH)], src_v)
    pltpu.sync_copy(dst_hbm.at[pl.ds(ebase + half * NH, NH)], dst_v)
    pltpu.async_copy(g_hbm.at[src_v.at[0]], bufs[0], sems[0])
    pltpu.async_copy(g_hbm.at[src_v.at[1]], bufs[1], sems[1])

    def body(i, _):
      c = i * 2
      for b in range(2):
        pltpu.make_async_copy(g_hbm.at[src_v.at[c + b]], bufs[b],
                              sems[b]).wait()
        pltpu.sync_copy(bufs[b], acc_sh.at[dst_v.at[c + b]], add=True)
        pltpu.async_copy(g_hbm.at[src_v.at[c + b + 2]], bufs[b], sems[b])
      return 0

    lax.fori_loop(0, NH // 2 - 1, body, 0)
    for b in range(2):
      c = NH - 2 + b
      pltpu.make_async_copy(g_hbm.at[src_v.at[c]], bufs[b], sems[b]).wait()
      pltpu.sync_copy(bufs[b], acc_sh.at[dst_v.at[c]], add=True)
  plsc.subcore_barrier()

  pltpu.sync_copy(acc_sh.at[pl.ds(row0, RPT)],
                  out_hbm.at[cid, pl.ds(row0, RPT)])


_BR = 2000
_GRID = N // _BR


def _dinv(d0, d1):
  deg = d0[:, 0:1] + d1[:, 0:1] + 1.0
  return lax.rsqrt(deg)


def _pre_body(x_ref, w_ref, d0_ref, d1_ref, o_ref):
  h = jnp.dot(x_ref[...], w_ref[...], preferred_element_type=jnp.float32)
  o_ref[...] = h * _dinv(d0_ref[...], d1_ref[...])


def _mid_body(a0_ref, a1_ref, g_ref, d0_ref, d1_ref, b_ref, w_ref, o_ref):
  dinv = _dinv(d0_ref[...], d1_ref[...])
  h = dinv * (a0_ref[...] + a1_ref[...] + g_ref[...]) + b_ref[...]
  h = jnp.maximum(h, 0.0)
  o_ref[...] = jnp.dot(h, w_ref[...],
                       preferred_element_type=jnp.float32) * dinv


def _post_body(a0_ref, a1_ref, g_ref, d0_ref, d1_ref, b_ref, o_ref):
  dinv = _dinv(d0_ref[...], d1_ref[...])
  o_ref[...] = dinv * (a0_ref[...] + a1_ref[...] + g_ref[...]) + b_ref[...]


_row_spec = pl.BlockSpec((_BR, D), lambda i: (i, 0))
_deg_spec = pl.BlockSpec((_BR, DEGW), lambda i: (i, 0))
_w_spec = pl.BlockSpec((D, D), lambda i: (0, 0))
_b_spec = pl.BlockSpec((1, D), lambda i: (0, 0))
_out_t = jax.ShapeDtypeStruct((N, D), jnp.float32)

_pre = pl.pallas_call(
    _pre_body, grid=(_GRID,),
    in_specs=[_row_spec, _w_spec, _deg_spec, _deg_spec],
    out_specs=_row_spec, out_shape=_out_t)

_mid = pl.pallas_call(
    _mid_body, grid=(_GRID,),
    in_specs=[_row_spec, _row_spec, _row_spec, _deg_spec, _deg_spec,
              _b_spec, _w_spec],
    out_specs=_row_spec, out_shape=_out_t)

_post = pl.pallas_call(
    _post_body, grid=(_GRID,),
    in_specs=[_row_spec, _row_spec, _row_spec, _deg_spec, _deg_spec, _b_spec],
    out_specs=_row_spec, out_shape=_out_t)


def kernel(x, edge_index, W1, b1, W2, b2):
  src = edge_index[0].astype(jnp.int32).reshape(NW * NCH, CH)
  dst = edge_index[1].astype(jnp.int32).reshape(NW * NCH, CH)
  b1 = b1.reshape(1, D)
  b2 = b2.reshape(1, D)

  deg_k, msg_k = _sc_kernels()
  degp = deg_k(dst)
  d0, d1 = degp[0], degp[1]
  g1 = _pre(x, W1, d0, d1)
  acc1 = msg_k(g1, src, dst)
  g2 = _mid(acc1[0], acc1[1], g1, d0, d1, b1, W2)
  acc2 = msg_k(g2, src, dst)
  return _post(acc2[0], acc2[1], g2, d0, d1, b2)
